# Initial kernel scaffold; baseline (speedup 1.0000x reference)
import jax
import jax.numpy as jnp
from jax.experimental import pallas as pl


def kernel(x, edge_index, W):
    raise NotImplementedError("write your pallas kernel here")


# speedup vs baseline: 6.0542x; 6.0542x over previous
import functools

import jax
import jax.numpy as jnp
from jax import lax
from jax.experimental import pallas as pl
from jax.experimental.pallas import tpu as pltpu
from jax.experimental.pallas import tpu_sc as plsc

N_NODES = 10000
N_EDGES = 320000
D = 128

NC = 2
NS = 16
NW = NC * NS

CH = 128
NCHUNK = N_EDGES // CH
MAX_G = (NCHUNK + NW - 1) // NW
SEG = 624
TAIL = N_NODES - NS * SEG


def _mm_body(x_ref, w_ref, o_ref):
    o_ref[...] = lax.dot_general(
        x_ref[...], w_ref[...], (((1,), (1,)), ((), ())),
        preferred_element_type=jnp.float32)


def _matmul(x, W):
    return pl.pallas_call(
        _mm_body,
        grid=(10,),
        in_specs=[
            pl.BlockSpec((1000, D), lambda i: (i, 0)),
            pl.BlockSpec((D, D), lambda i: (0, 0)),
        ],
        out_specs=pl.BlockSpec((1000, D), lambda i: (i, 0)),
        out_shape=jax.ShapeDtypeStruct((N_NODES, D), jnp.float32),
    )(x, W)


def _agg_body(h_hbm, src_hbm, dst_hbm, out_hbm, src_v, dst_v, rows_v, y_sh):
    c = lax.axis_index("c")
    s = lax.axis_index("s")
    wid = c * NS + s

    zeros16 = jnp.zeros((16,), jnp.float32)

    def zrow(i, _):
        for j in range(D // 16):
            rows_v[i, pl.ds(j * 16, 16)] = zeros16
        return 0

    lax.fori_loop(0, CH, zrow, 0)

    base_row = s * SEG
    for k in range(SEG // CH):
        pltpu.sync_copy(rows_v, y_sh.at[pl.ds(base_row + k * CH, CH)])
    rem = SEG % CH
    pltpu.sync_copy(rows_v.at[pl.ds(0, rem)],
                    y_sh.at[pl.ds(base_row + (SEG // CH) * CH, rem)])

    @pl.when(s == NS - 1)
    def _():
        pltpu.sync_copy(rows_v.at[pl.ds(0, TAIL)],
                        y_sh.at[pl.ds(NS * SEG, TAIL)])

    plsc.subcore_barrier()

    def body(g, _):
        ch = wid + g * NW

        @pl.when(ch < NCHUNK)
        def _():
            base = ch * CH
            pltpu.sync_copy(src_hbm.at[pl.ds(base, CH)], src_v)
            pltpu.sync_copy(dst_hbm.at[pl.ds(base, CH)], dst_v)
            pltpu.sync_copy(h_hbm.at[src_v], rows_v)
            pltpu.sync_copy(rows_v, y_sh.at[dst_v], add=True)

        return 0

    lax.fori_loop(0, MAX_G, body, 0)

    plsc.subcore_barrier()

    pltpu.sync_copy(y_sh.at[pl.ds(base_row, SEG)],
                    out_hbm.at[c, pl.ds(base_row, SEG)])

    @pl.when(s == NS - 1)
    def _():
        pltpu.sync_copy(y_sh.at[pl.ds(NS * SEG, TAIL)],
                        out_hbm.at[c, pl.ds(NS * SEG, TAIL)])


def _aggregate(h, src, dst):
    mesh = plsc.VectorSubcoreMesh(
        core_axis_name="c", subcore_axis_name="s", num_cores=NC,
        num_subcores=NS)
    f = pl.kernel(
        _agg_body,
        out_type=jax.ShapeDtypeStruct((NC, N_NODES, D), jnp.float32),
        mesh=mesh,
        scratch_types=[
            pltpu.VMEM((CH,), jnp.int32),
            pltpu.VMEM((CH,), jnp.int32),
            pltpu.VMEM((CH, D), jnp.float32),
            pltpu.VMEM_SHARED((N_NODES, D), jnp.float32),
        ],
    )
    return f(h, src, dst)


def _add_body(a_ref, o_ref):
    o_ref[...] = a_ref[0] + a_ref[1]


def _combine(reps):
    return pl.pallas_call(
        _add_body,
        grid=(10,),
        in_specs=[pl.BlockSpec((NC, 1000, D), lambda i: (0, i, 0))],
        out_specs=pl.BlockSpec((1000, D), lambda i: (i, 0)),
        out_shape=jax.ShapeDtypeStruct((N_NODES, D), jnp.float32),
    )(reps)


def kernel(x, edge_index, W):
    h = _matmul(x, W)
    reps = _aggregate(h, edge_index[0], edge_index[1])
    return _combine(reps)
